# baseline (speedup 1.0000x reference)
import jax
import jax.numpy as jnp
from jax import lax
from jax.experimental import pallas as pl
from jax.experimental.pallas import tpu as pltpu
from jax.experimental.pallas import tpu_sc as plsc
from functools import partial

BIN_COUNT = 128
MIN_VALUE = -0.49
MAX_VALUE = 0.49

N_SC = 256
BN = 128
NW = 32
CH = 256


def _bins_body(emb_ref, bins_ref):
    emb = emb_ref[...]
    norm = jnp.sqrt(jnp.sum(emb * emb, axis=1, keepdims=True))
    normed = emb / jnp.maximum(norm, 1e-12)
    bin_f = (normed - MIN_VALUE) * BIN_COUNT / (MAX_VALUE - MIN_VALUE)
    bin_f = jnp.clip(bin_f, 0.0, BIN_COUNT - 1)
    bins_ref[...] = bin_f.astype(jnp.int32)


def _tc_body(x_ref, bins_ref, out_ref, *, bn, c, b):
    i = pl.program_id(0)
    x = x_ref[...]
    s = jnp.sum(jnp.exp(x), axis=2)
    lse = jnp.log(s)

    iota = jax.lax.broadcasted_iota(jnp.int32, (bn, c, b), 2)
    xb = jnp.where(bins_ref[...][:, :, None] == iota, x, 0.0)
    partial_sum = jnp.sum(xb) - jnp.sum(lse)

    @pl.when(i == 0)
    def _init():
        out_ref[...] = jnp.zeros((1, 1), jnp.float32)

    out_ref[...] += partial_sum.reshape(1, 1)


def _sc_body(lf_hbm, bins_hbm, out_s, out_p, xbuf, binbuf, sbuf, pbuf,
             *, row0, rows_w, b):
    wid = lax.axis_index("s") * 2 + lax.axis_index("c")
    base = row0 + wid * rows_w
    lanes = lax.iota(jnp.int32, 16)
    n_chunks = rows_w // CH

    pick_acc = jnp.zeros((16,), jnp.float32)
    for g in range(n_chunks):
        chunk_row = base + g * CH
        pltpu.sync_copy(lf_hbm.at[pl.ds(chunk_row * b, CH * b)], xbuf)
        pltpu.sync_copy(bins_hbm.at[pl.ds(chunk_row, CH)], binbuf)

        def tile_body(t, acc):
            row_idx = lanes * b + t * (16 * b)
            binv = binbuf[pl.ds(t * 16, 16)]
            pick = plsc.load_gather(xbuf, [row_idx + binv])
            accs = [jnp.zeros((16,), jnp.float32) for _ in range(8)]
            for col in range(0, b, 8):
                for j in range(8):
                    accs[j] = accs[j] + jnp.exp(
                        plsc.load_gather(xbuf, [row_idx + (col + j)]))
            s_acc = ((accs[0] + accs[1]) + (accs[2] + accs[3])) + (
                (accs[4] + accs[5]) + (accs[6] + accs[7]))
            sbuf[pl.ds(t * 16, 16)] = s_acc
            return acc + pick

        pick_acc = lax.fori_loop(0, CH // 16, tile_body, pick_acc)
        pltpu.sync_copy(sbuf, out_s.at[pl.ds(wid * rows_w + g * CH, CH)])

    pbuf[...] = pick_acc
    pltpu.sync_copy(pbuf, out_p.at[pl.ds(wid * 16, 16)])


def _final_body(tcp_ref, s_ref, p_ref, out_ref, *, denom):
    lse_sum = jnp.sum(jnp.log(s_ref[...]))
    pick_sum = jnp.sum(p_ref[...])
    total = tcp_ref[...] + (pick_sum - lse_sum).reshape(1, 1)
    out_ref[...] = -total / denom


def kernel(logits, teachor_embeddings, labels):
    del labels
    N, C, B = logits.shape
    n_tc = N - N_SC
    rows_sc = N_SC * C
    rows_w = rows_sc // NW

    bins = pl.pallas_call(
        _bins_body,
        out_shape=jax.ShapeDtypeStruct((N, C), jnp.int32),
    )(teachor_embeddings)

    tc_partial = pl.pallas_call(
        partial(_tc_body, bn=BN, c=C, b=B),
        grid=(n_tc // BN,),
        in_specs=[
            pl.BlockSpec((BN, C, B), lambda i: (i, 0, 0)),
            pl.BlockSpec((BN, C), lambda i: (i, 0)),
        ],
        out_specs=pl.BlockSpec((1, 1), lambda i: (0, 0)),
        out_shape=jax.ShapeDtypeStruct((1, 1), jnp.float32),
    )(logits, bins)

    logits_flat = logits.reshape(N * C * B)
    bins_flat = bins.reshape(N * C)

    mesh = plsc.VectorSubcoreMesh(core_axis_name="c", subcore_axis_name="s")
    sc_sums, sc_picks = pl.kernel(
        partial(_sc_body, row0=n_tc * C, rows_w=rows_w, b=B),
        out_type=[
            jax.ShapeDtypeStruct((rows_sc,), jnp.float32),
            jax.ShapeDtypeStruct((NW * 16,), jnp.float32),
        ],
        mesh=mesh,
        compiler_params=pltpu.CompilerParams(needs_layout_passes=False),
        scratch_types=[
            pltpu.VMEM((CH * B,), jnp.float32),
            pltpu.VMEM((CH,), jnp.int32),
            pltpu.VMEM((CH,), jnp.float32),
            pltpu.VMEM((16,), jnp.float32),
        ],
    )(logits_flat, bins_flat)

    out = pl.pallas_call(
        partial(_final_body, denom=float(N * C)),
        in_specs=[
            pl.BlockSpec((1, 1), lambda: (0, 0)),
            pl.BlockSpec((N_SC, C), lambda: (0, 0)),
            pl.BlockSpec((NW, 16), lambda: (0, 0)),
        ],
        out_specs=pl.BlockSpec((1, 1), lambda: (0, 0)),
        out_shape=jax.ShapeDtypeStruct((1, 1), jnp.float32),
    )(tc_partial, sc_sums.reshape(N_SC, C), sc_picks.reshape(NW, 16))
    return out[0, 0]


# speedup vs baseline: 1.7616x; 1.7616x over previous
import jax
import jax.numpy as jnp
from jax import lax
from jax.experimental import pallas as pl
from jax.experimental.pallas import tpu as pltpu
from jax.experimental.pallas import tpu_sc as plsc
from functools import partial

BIN_COUNT = 128
MIN_VALUE = -0.49
MAX_VALUE = 0.49

N_SC = 256
BN = 128
NW = 32
CH = 256


def _bins_body(emb_ref, bins_ref):
    emb = emb_ref[...]
    norm = jnp.sqrt(jnp.sum(emb * emb, axis=1, keepdims=True))
    normed = emb / jnp.maximum(norm, 1e-12)
    bin_f = (normed - MIN_VALUE) * BIN_COUNT / (MAX_VALUE - MIN_VALUE)
    bin_f = jnp.clip(bin_f, 0.0, BIN_COUNT - 1)
    bins_ref[...] = bin_f.astype(jnp.int32)


def _tc_body(x_ref, bins_ref, out_ref, *, bn, c, b):
    i = pl.program_id(0)
    x = x_ref[...]
    s = jnp.sum(jnp.exp(x), axis=2)
    lse = jnp.log(s)

    iota = jax.lax.broadcasted_iota(jnp.int32, (bn, c, b), 2)
    xb = jnp.where(bins_ref[...][:, :, None] == iota, x, 0.0)
    partial_sum = jnp.sum(xb) - jnp.sum(lse)

    @pl.when(i == 0)
    def _init():
        out_ref[...] = jnp.zeros((1, 1), jnp.float32)

    out_ref[...] += partial_sum.reshape(1, 1)


def _sc_body(lf_hbm, bins_hbm, out_s, out_p, xbuf, binbuf, sbuf, pbuf,
             *, row0, rows_w, b):
    wid = lax.axis_index("s") * 2 + lax.axis_index("c")
    base = row0 + wid * rows_w
    lanes = lax.iota(jnp.int32, 16)
    n_chunks = rows_w // CH

    pick_acc = jnp.zeros((16,), jnp.float32)
    for g in range(n_chunks):
        chunk_row = base + g * CH
        pltpu.sync_copy(lf_hbm.at[pl.ds(chunk_row * b, CH * b)], xbuf)
        pltpu.sync_copy(bins_hbm.at[pl.ds(chunk_row, CH)], binbuf)

        def tile_body(t, acc):
            row_idx = lanes * b + t * (16 * b)
            binv = binbuf[pl.ds(t * 16, 16)]
            pick = plsc.load_gather(xbuf, [row_idx + binv])
            accs = [jnp.zeros((16,), jnp.float32) for _ in range(8)]
            for col in range(0, b, 8):
                for j in range(8):
                    cvec = (lanes + (col + j)) & (b - 1)
                    accs[j] = accs[j] + jnp.exp(
                        plsc.load_gather(xbuf, [row_idx + cvec]))
            s_acc = ((accs[0] + accs[1]) + (accs[2] + accs[3])) + (
                (accs[4] + accs[5]) + (accs[6] + accs[7]))
            sbuf[pl.ds(t * 16, 16)] = s_acc
            return acc + pick

        pick_acc = lax.fori_loop(0, CH // 16, tile_body, pick_acc)
        pltpu.sync_copy(sbuf, out_s.at[pl.ds(wid * rows_w + g * CH, CH)])

    pbuf[...] = pick_acc
    pltpu.sync_copy(pbuf, out_p.at[pl.ds(wid * 16, 16)])


def _final_body(tcp_ref, s_ref, p_ref, out_ref, *, denom):
    lse_sum = jnp.sum(jnp.log(s_ref[...]))
    pick_sum = jnp.sum(p_ref[...])
    total = tcp_ref[...] + (pick_sum - lse_sum).reshape(1, 1)
    out_ref[...] = -total / denom


def kernel(logits, teachor_embeddings, labels):
    del labels
    N, C, B = logits.shape
    n_tc = N - N_SC
    rows_sc = N_SC * C
    rows_w = rows_sc // NW

    bins = pl.pallas_call(
        _bins_body,
        out_shape=jax.ShapeDtypeStruct((N, C), jnp.int32),
    )(teachor_embeddings)

    tc_partial = pl.pallas_call(
        partial(_tc_body, bn=BN, c=C, b=B),
        grid=(n_tc // BN,),
        in_specs=[
            pl.BlockSpec((BN, C, B), lambda i: (i, 0, 0)),
            pl.BlockSpec((BN, C), lambda i: (i, 0)),
        ],
        out_specs=pl.BlockSpec((1, 1), lambda i: (0, 0)),
        out_shape=jax.ShapeDtypeStruct((1, 1), jnp.float32),
    )(logits, bins)

    logits_flat = logits.reshape(N * C * B)
    bins_flat = bins.reshape(N * C)

    mesh = plsc.VectorSubcoreMesh(core_axis_name="c", subcore_axis_name="s")
    sc_sums, sc_picks = pl.kernel(
        partial(_sc_body, row0=n_tc * C, rows_w=rows_w, b=B),
        out_type=[
            jax.ShapeDtypeStruct((rows_sc,), jnp.float32),
            jax.ShapeDtypeStruct((NW * 16,), jnp.float32),
        ],
        mesh=mesh,
        compiler_params=pltpu.CompilerParams(needs_layout_passes=False),
        scratch_types=[
            pltpu.VMEM((CH * B,), jnp.float32),
            pltpu.VMEM((CH,), jnp.int32),
            pltpu.VMEM((CH,), jnp.float32),
            pltpu.VMEM((16,), jnp.float32),
        ],
    )(logits_flat, bins_flat)

    out = pl.pallas_call(
        partial(_final_body, denom=float(N * C)),
        in_specs=[
            pl.BlockSpec((1, 1), lambda: (0, 0)),
            pl.BlockSpec((N_SC, C), lambda: (0, 0)),
            pl.BlockSpec((NW, 16), lambda: (0, 0)),
        ],
        out_specs=pl.BlockSpec((1, 1), lambda: (0, 0)),
        out_shape=jax.ShapeDtypeStruct((1, 1), jnp.float32),
    )(tc_partial, sc_sums.reshape(N_SC, C), sc_picks.reshape(NW, 16))
    return out[0, 0]


# speedup vs baseline: 1.7639x; 1.0013x over previous
import jax
import jax.numpy as jnp
from jax import lax
from jax.experimental import pallas as pl
from jax.experimental.pallas import tpu as pltpu
from jax.experimental.pallas import tpu_sc as plsc
from functools import partial

BIN_COUNT = 128
MIN_VALUE = -0.49
MAX_VALUE = 0.49

N_SC = 256
BN = 128
NW = 32
CH = 256


def _bins_body(emb_ref, bins_ref):
    emb = emb_ref[...]
    norm = jnp.sqrt(jnp.sum(emb * emb, axis=1, keepdims=True))
    normed = emb / jnp.maximum(norm, 1e-12)
    bin_f = (normed - MIN_VALUE) * BIN_COUNT / (MAX_VALUE - MIN_VALUE)
    bin_f = jnp.clip(bin_f, 0.0, BIN_COUNT - 1)
    bins_ref[...] = bin_f.astype(jnp.int32)


def _tc_body(x_ref, bins_ref, out_ref, *, bn, c, b):
    i = pl.program_id(0)
    x = x_ref[...]
    s = jnp.sum(jnp.exp(x), axis=2)
    lse = jnp.log(s)

    iota = jax.lax.broadcasted_iota(jnp.int32, (bn, c, b), 2)
    xb = jnp.where(bins_ref[...][:, :, None] == iota, x, 0.0)
    partial_sum = jnp.sum(xb) - jnp.sum(lse)

    @pl.when(i == 0)
    def _init():
        out_ref[...] = jnp.zeros((1, 1), jnp.float32)

    out_ref[...] += partial_sum.reshape(1, 1)


def _sc_body(lf_hbm, bins_hbm, out_s, out_p, xbuf, binbuf, sbuf, pbuf,
             *, row0, rows_w, b):
    wid = lax.axis_index("s") * 2 + lax.axis_index("c")
    base = row0 + wid * rows_w
    lanes = lax.iota(jnp.int32, 16)
    n_chunks = rows_w // CH

    pick_acc = jnp.zeros((16,), jnp.float32)
    for g in range(n_chunks):
        chunk_row = base + g * CH
        pltpu.sync_copy(lf_hbm.at[pl.ds(chunk_row * b, CH * b)], xbuf)
        pltpu.sync_copy(bins_hbm.at[pl.ds(chunk_row, CH)], binbuf)

        def tile_body(t, acc):
            row_idx = lanes * b + t * (16 * b)
            binv = binbuf[pl.ds(t * 16, 16)]
            pick = plsc.load_gather(xbuf, [row_idx + binv])
            accs = [jnp.zeros((16,), jnp.float32) for _ in range(8)]
            for col in range(0, b, 8):
                for j in range(8):
                    cvec = (lanes + (col + j)) & (b - 1)
                    accs[j] = accs[j] + jnp.exp(
                        plsc.load_gather(xbuf, [row_idx + cvec]))
            s_acc = ((accs[0] + accs[1]) + (accs[2] + accs[3])) + (
                (accs[4] + accs[5]) + (accs[6] + accs[7]))
            sbuf[pl.ds(t * 16, 16)] = s_acc
            return acc + pick

        pick_acc = lax.fori_loop(0, CH // 16, tile_body, pick_acc)
        pltpu.sync_copy(sbuf, out_s.at[pl.ds(wid * rows_w + g * CH, CH)])

    pbuf[...] = pick_acc
    pltpu.sync_copy(pbuf, out_p.at[pl.ds(wid * 16, 16)])


def _final_body(tcp_ref, s_ref, p_ref, out_ref, *, denom):
    lse_sum = jnp.sum(jnp.log(s_ref[...]))
    pick_sum = jnp.sum(p_ref[...])
    total = tcp_ref[...] + (pick_sum - lse_sum).reshape(1, 1)
    out_ref[...] = -total / denom


def kernel(logits, teachor_embeddings, labels):
    del labels
    N, C, B = logits.shape
    n_tc = N - N_SC
    rows_sc = N_SC * C
    rows_w = rows_sc // NW

    bins = pl.pallas_call(
        _bins_body,
        out_shape=jax.ShapeDtypeStruct((N, C), jnp.int32),
    )(teachor_embeddings)

    logits_flat = logits.reshape(N * C * B)
    bins_flat = bins.reshape(N * C)

    mesh = plsc.VectorSubcoreMesh(core_axis_name="c", subcore_axis_name="s")
    sc_sums, sc_picks = pl.kernel(
        partial(_sc_body, row0=n_tc * C, rows_w=rows_w, b=B),
        out_type=[
            jax.ShapeDtypeStruct((rows_sc,), jnp.float32),
            jax.ShapeDtypeStruct((NW * 16,), jnp.float32),
        ],
        mesh=mesh,
        compiler_params=pltpu.CompilerParams(needs_layout_passes=False),
        scratch_types=[
            pltpu.VMEM((CH * B,), jnp.float32),
            pltpu.VMEM((CH,), jnp.int32),
            pltpu.VMEM((CH,), jnp.float32),
            pltpu.VMEM((16,), jnp.float32),
        ],
    )(logits_flat, bins_flat)

    tc_partial = pl.pallas_call(
        partial(_tc_body, bn=BN, c=C, b=B),
        grid=(n_tc // BN,),
        in_specs=[
            pl.BlockSpec((BN, C, B), lambda i: (i, 0, 0)),
            pl.BlockSpec((BN, C), lambda i: (i, 0)),
        ],
        out_specs=pl.BlockSpec((1, 1), lambda i: (0, 0)),
        out_shape=jax.ShapeDtypeStruct((1, 1), jnp.float32),
    )(logits, bins)


    out = pl.pallas_call(
        partial(_final_body, denom=float(N * C)),
        in_specs=[
            pl.BlockSpec((1, 1), lambda: (0, 0)),
            pl.BlockSpec((N_SC, C), lambda: (0, 0)),
            pl.BlockSpec((NW, 16), lambda: (0, 0)),
        ],
        out_specs=pl.BlockSpec((1, 1), lambda: (0, 0)),
        out_shape=jax.ShapeDtypeStruct((1, 1), jnp.float32),
    )(tc_partial, sc_sums.reshape(N_SC, C), sc_picks.reshape(NW, 16))
    return out[0, 0]


# speedup vs baseline: 1.9879x; 1.1270x over previous
import jax
import jax.numpy as jnp
from jax import lax
from jax.experimental import pallas as pl
from jax.experimental.pallas import tpu as pltpu
from jax.experimental.pallas import tpu_sc as plsc
from functools import partial

BIN_COUNT = 128
MIN_VALUE = -0.49
MAX_VALUE = 0.49

N_SC = 128
BN = 128
NW = 32
CH = 256


def _bins_body(emb_ref, bins_ref):
    emb = emb_ref[...]
    norm = jnp.sqrt(jnp.sum(emb * emb, axis=1, keepdims=True))
    normed = emb / jnp.maximum(norm, 1e-12)
    bin_f = (normed - MIN_VALUE) * BIN_COUNT / (MAX_VALUE - MIN_VALUE)
    bin_f = jnp.clip(bin_f, 0.0, BIN_COUNT - 1)
    bins_ref[...] = bin_f.astype(jnp.int32)


def _tc_body(x_ref, bins_ref, out_ref, *, bn, c, b):
    i = pl.program_id(0)
    x = x_ref[...]
    s = jnp.sum(jnp.exp(x), axis=2)
    lse = jnp.log(s)

    iota = jax.lax.broadcasted_iota(jnp.int32, (bn, c, b), 2)
    xb = jnp.where(bins_ref[...][:, :, None] == iota, x, 0.0)
    partial_sum = jnp.sum(xb) - jnp.sum(lse)

    @pl.when(i == 0)
    def _init():
        out_ref[...] = jnp.zeros((1, 1), jnp.float32)

    out_ref[...] += partial_sum.reshape(1, 1)


def _sc_body(lf_hbm, bins_hbm, out_s, out_p, xbuf, binbuf, sbuf, pbuf,
             *, row0, rows_w, b):
    wid = lax.axis_index("s") * 2 + lax.axis_index("c")
    base = row0 + wid * rows_w
    lanes = lax.iota(jnp.int32, 16)
    n_chunks = rows_w // CH

    pick_acc = jnp.zeros((16,), jnp.float32)
    for g in range(n_chunks):
        chunk_row = base + g * CH
        pltpu.sync_copy(lf_hbm.at[pl.ds(chunk_row * b, CH * b)], xbuf)
        pltpu.sync_copy(bins_hbm.at[pl.ds(chunk_row, CH)], binbuf)

        def tile_body(t, acc):
            row_idx = lanes * b + t * (16 * b)
            binv = binbuf[pl.ds(t * 16, 16)]
            pick = plsc.load_gather(xbuf, [row_idx + binv])
            accs = [jnp.zeros((16,), jnp.float32) for _ in range(8)]
            for col in range(0, b, 8):
                for j in range(8):
                    cvec = (lanes + (col + j)) & (b - 1)
                    accs[j] = accs[j] + jnp.exp(
                        plsc.load_gather(xbuf, [row_idx + cvec]))
            s_acc = ((accs[0] + accs[1]) + (accs[2] + accs[3])) + (
                (accs[4] + accs[5]) + (accs[6] + accs[7]))
            sbuf[pl.ds(t * 16, 16)] = s_acc
            return acc + pick

        pick_acc = lax.fori_loop(0, CH // 16, tile_body, pick_acc)
        pltpu.sync_copy(sbuf, out_s.at[pl.ds(wid * rows_w + g * CH, CH)])

    pbuf[...] = pick_acc
    pltpu.sync_copy(pbuf, out_p.at[pl.ds(wid * 16, 16)])


def _final_body(tcp_ref, s_ref, p_ref, out_ref, *, denom):
    lse_sum = jnp.sum(jnp.log(s_ref[...]))
    pick_sum = jnp.sum(p_ref[...])
    total = tcp_ref[...] + (pick_sum - lse_sum).reshape(1, 1)
    out_ref[...] = -total / denom


def kernel(logits, teachor_embeddings, labels):
    del labels
    N, C, B = logits.shape
    n_tc = N - N_SC
    rows_sc = N_SC * C
    rows_w = rows_sc // NW

    bins = pl.pallas_call(
        _bins_body,
        out_shape=jax.ShapeDtypeStruct((N, C), jnp.int32),
    )(teachor_embeddings)

    logits_flat = logits.reshape(N * C * B)
    bins_flat = bins.reshape(N * C)

    mesh = plsc.VectorSubcoreMesh(core_axis_name="c", subcore_axis_name="s")
    sc_sums, sc_picks = pl.kernel(
        partial(_sc_body, row0=n_tc * C, rows_w=rows_w, b=B),
        out_type=[
            jax.ShapeDtypeStruct((rows_sc,), jnp.float32),
            jax.ShapeDtypeStruct((NW * 16,), jnp.float32),
        ],
        mesh=mesh,
        compiler_params=pltpu.CompilerParams(needs_layout_passes=False),
        scratch_types=[
            pltpu.VMEM((CH * B,), jnp.float32),
            pltpu.VMEM((CH,), jnp.int32),
            pltpu.VMEM((CH,), jnp.float32),
            pltpu.VMEM((16,), jnp.float32),
        ],
    )(logits_flat, bins_flat)

    tc_partial = pl.pallas_call(
        partial(_tc_body, bn=BN, c=C, b=B),
        grid=(n_tc // BN,),
        in_specs=[
            pl.BlockSpec((BN, C, B), lambda i: (i, 0, 0)),
            pl.BlockSpec((BN, C), lambda i: (i, 0)),
        ],
        out_specs=pl.BlockSpec((1, 1), lambda i: (0, 0)),
        out_shape=jax.ShapeDtypeStruct((1, 1), jnp.float32),
    )(logits, bins)


    out = pl.pallas_call(
        partial(_final_body, denom=float(N * C)),
        in_specs=[
            pl.BlockSpec((1, 1), lambda: (0, 0)),
            pl.BlockSpec((N_SC, C), lambda: (0, 0)),
            pl.BlockSpec((NW, 16), lambda: (0, 0)),
        ],
        out_specs=pl.BlockSpec((1, 1), lambda: (0, 0)),
        out_shape=jax.ShapeDtypeStruct((1, 1), jnp.float32),
    )(tc_partial, sc_sums.reshape(N_SC, C), sc_picks.reshape(NW, 16))
    return out[0, 0]


# speedup vs baseline: 2.6288x; 1.3224x over previous
import jax
import jax.numpy as jnp
from jax.experimental import pallas as pl
from functools import partial

BIN_COUNT = 128
MIN_VALUE = -0.49
MAX_VALUE = 0.49


def _bins_body(emb_ref, bins_ref):
    emb = emb_ref[...]
    norm = jnp.sqrt(jnp.sum(emb * emb, axis=1, keepdims=True))
    normed = emb / jnp.maximum(norm, 1e-12)
    bin_f = (normed - MIN_VALUE) * BIN_COUNT / (MAX_VALUE - MIN_VALUE)
    bin_f = jnp.clip(bin_f, 0.0, BIN_COUNT - 1)
    bins_ref[...] = bin_f.astype(jnp.int32)


def _loss_body(x_ref, bins_ref, out_ref, *, bn, c, b, n_blocks, denom):
    i = pl.program_id(0)
    x = x_ref[...]
    s = jnp.sum(jnp.exp(x), axis=2)
    lse = jnp.log(s)

    iota = jax.lax.broadcasted_iota(jnp.int32, (bn, c, b), 2)
    xb = jnp.where(bins_ref[...][:, :, None] == iota, x, 0.0)
    partial_sum = jnp.sum(xb) - jnp.sum(lse)

    @pl.when(i == 0)
    def _init():
        out_ref[...] = jnp.zeros((1, 1), jnp.float32)

    out_ref[...] += partial_sum.reshape(1, 1)

    @pl.when(i == n_blocks - 1)
    def _fini():
        out_ref[...] = -out_ref[...] / denom


def kernel(logits, teachor_embeddings, labels):
    del labels
    N, C, B = logits.shape

    bins = pl.pallas_call(
        _bins_body,
        out_shape=jax.ShapeDtypeStruct((N, C), jnp.int32),
    )(teachor_embeddings)
    BN = 128
    n_blocks = N // BN

    out = pl.pallas_call(
        partial(_loss_body, bn=BN, c=C, b=B, n_blocks=n_blocks,
                denom=float(N * C)),
        grid=(n_blocks,),
        in_specs=[
            pl.BlockSpec((BN, C, B), lambda i: (i, 0, 0)),
            pl.BlockSpec((BN, C), lambda i: (i, 0)),
        ],
        out_specs=pl.BlockSpec((1, 1), lambda i: (0, 0)),
        out_shape=jax.ShapeDtypeStruct((1, 1), jnp.float32),
    )(logits, bins)
    return out[0, 0]
